# baseline (speedup 1.0000x reference)
import jax
import jax.numpy as jnp
from jax import lax
from jax.experimental import pallas as pl
from jax.experimental.pallas import tpu as pltpu
from jax.experimental.pallas import tpu_sc as plsc

N = 10000
D = 128
HID = 256
CLF_H = 64
OUT = 3
EPS = 1e-5
F32 = jnp.float32

NS = 16
RPC = 128
E_EDGES = 320000
RT = 2560
L0_RT = RT // NS
POOL_RT = RT // 2 // NS
SG = 40
NSINK = N + 8
ROWS_PT = 624
TAIL = N - NS * ROWS_PT


def _zero_vmem(ref, nrows, width):
    z = jnp.zeros((16,), F32)

    def body(i, _):
        for k in range(width // 16):
            ref[i, pl.ds(k * 16, 16)] = z
        return 0

    lax.fori_loop(0, nrows, body, 0)


def _fill_vmem(ref, nrows, width, val):
    v = jnp.full((16,), val, F32)

    def body(i, _):
        for k in range(width // 16):
            ref[i, pl.ds(k * 16, 16)] = v
        return 0

    lax.fori_loop(0, nrows, body, 0)


def _copy_rows(src, dst, dst_base, nrows, chunk):
    full, rem = divmod(nrows, chunk)
    for t in range(full):
        pltpu.sync_copy(src, dst.at[pl.ds(dst_base + t * chunk, chunk), :])
    if rem:
        pltpu.sync_copy(src.at[pl.ds(0, rem), :],
                        dst.at[pl.ds(dst_base + full * chunk, rem), :])


def _zero_own_rows(acc, zsrc, s):
    _copy_rows(zsrc, acc, s * ROWS_PT, ROWS_PT, zsrc.shape[0])
    pl.when(s == NS - 1)(lambda: pltpu.sync_copy(
        zsrc.at[pl.ds(0, TAIL), :], acc.at[pl.ds(N - TAIL, TAIL), :]))


def _dump_own_rows(acc, out, s):
    base = s * ROWS_PT
    pltpu.sync_copy(acc.at[pl.ds(base, ROWS_PT), :],
                    out.at[pl.ds(base, ROWS_PT), :])
    pl.when(s == NS - 1)(lambda: pltpu.sync_copy(
        acc.at[pl.ds(N - TAIL, TAIL), :], out.at[pl.ds(N - TAIL, TAIL), :]))


def _seg_stream(tab, src_t, dst_t, acc, rows_a, rows_b, sem_a, sem_b, nrows):
    del rows_b, sem_b

    def body(j, _):
        pltpu.async_copy(tab.at[src_t.at[j]], rows_a, sem_a).wait()
        pltpu.sync_copy(rows_a, acc.at[dst_t.at[j]], add=True)
        return 0

    lax.fori_loop(0, nrows, body, 0)


def _stream_groups(tab, src2, dst2, base, n_groups, acc,
                   src_t, dst_t, rows_a, rows_b, sem_a, sem_b):
    def gbody(g, _):
        gb = base + g * SG
        pltpu.sync_copy(src2.at[pl.ds(gb, SG), :], src_t)
        pltpu.sync_copy(dst2.at[pl.ds(gb, SG), :], dst_t)
        _seg_stream(tab, src_t, dst_t, acc, rows_a, rows_b, sem_a, sem_b, SG)
        return 0

    lax.fori_loop(0, n_groups, gbody, 0)


def _sc_l0_body(x_hbm, src2, dst2, s_on_o, s_vo_o, c_on_o, c_vo_o,
                acc, src_t, dst_t, rows_a, rows_b, sem_a, sem_b):
    c = lax.axis_index("c")
    s = lax.axis_index("s")
    base = c * RT + s * L0_RT

    _zero_vmem(rows_a, RPC, D)
    _zero_own_rows(acc, rows_a, s)
    plsc.subcore_barrier()
    _stream_groups(x_hbm, src2, dst2, base, L0_RT // SG, acc,
                   src_t, dst_t, rows_a, rows_b, sem_a, sem_b)
    plsc.subcore_barrier()
    pl.when(c == 0)(lambda: _dump_own_rows(acc, s_on_o, s))
    pl.when(c == 1)(lambda: _dump_own_rows(acc, s_vo_o, s))

    _zero_vmem(rows_a, RPC, D)
    _zero_own_rows(acc, rows_a, s)
    _fill_vmem(rows_a, RPC, D, 1.0)
    plsc.subcore_barrier()

    def cgroup(g, _):
        pltpu.sync_copy(dst2.at[pl.ds(base + g * SG, SG), :], dst_t)

        def cbody(j, _):
            pltpu.sync_copy(rows_a, acc.at[dst_t.at[j]], add=True)
            return 0

        lax.fori_loop(0, SG, cbody, 0)
        return 0

    lax.fori_loop(0, L0_RT // SG, cgroup, 0)
    plsc.subcore_barrier()
    pl.when(c == 0)(lambda: _dump_own_rows(acc, c_on_o, s))
    pl.when(c == 1)(lambda: _dump_own_rows(acc, c_vo_o, s))


def _sc_l1_body(h0_hbm, h1_hbm, src2, dst2, on0_o, on1_o, vo0_o, vo1_o,
                acc, src_t, dst_t, rows_a, rows_b, sem_a, sem_b):
    c = lax.axis_index("c")
    s = lax.axis_index("s")

    for task, (out0, out1) in enumerate(((on0_o, on1_o), (vo0_o, vo1_o))):
        base = task * RT + s * L0_RT
        _zero_vmem(rows_a, RPC, D)
        _zero_own_rows(acc, rows_a, s)
        plsc.subcore_barrier()
        pl.when(c == 0)(lambda b=base: _stream_groups(
            h0_hbm, src2, dst2, b, L0_RT // SG, acc,
            src_t, dst_t, rows_a, rows_b, sem_a, sem_b))
        pl.when(c == 1)(lambda b=base: _stream_groups(
            h1_hbm, src2, dst2, b, L0_RT // SG, acc,
            src_t, dst_t, rows_a, rows_b, sem_a, sem_b))
        plsc.subcore_barrier()
        pl.when(c == 0)(lambda o=out0: _dump_own_rows(acc, o, s))
        pl.when(c == 1)(lambda o=out1: _dump_own_rows(acc, o, s))


def _sc_pool_body(h_hbm, src2, dst2, p0_o, p1_o,
                  acc, src_t, dst_t, rows_a, rows_b, sem_a, sem_b):
    c = lax.axis_index("c")
    s = lax.axis_index("s")
    base = c * (RT // 2) + s * POOL_RT

    _zero_vmem(rows_a, RPC, D)
    _zero_own_rows(acc, rows_a, s)
    plsc.subcore_barrier()
    _stream_groups(h_hbm, src2, dst2, base, POOL_RT // SG, acc,
                   src_t, dst_t, rows_a, rows_b, sem_a, sem_b)
    plsc.subcore_barrier()
    pl.when(c == 0)(lambda: _dump_own_rows(acc, p0_o, s))
    pl.when(c == 1)(lambda: _dump_own_rows(acc, p1_o, s))


def _make_sc_kernels():
    mesh = plsc.VectorSubcoreMesh(core_axis_name="c", subcore_axis_name="s",
                                  num_cores=2, num_subcores=NS)
    f = jax.ShapeDtypeStruct
    nd = f((N, D), F32)
    i32 = jnp.int32

    scratch = [
        pltpu.VMEM_SHARED((NSINK, D), F32),
        pltpu.VMEM((SG, RPC), i32), pltpu.VMEM((SG, RPC), i32),
        pltpu.VMEM((RPC, D), F32), pltpu.VMEM((RPC, D), F32),
        pltpu.SemaphoreType.DMA, pltpu.SemaphoreType.DMA,
    ]

    l0 = pl.kernel(_sc_l0_body, out_type=(nd, nd, nd, nd), mesh=mesh,
                   scratch_types=list(scratch))
    l1 = pl.kernel(_sc_l1_body, out_type=(nd, nd, nd, nd), mesh=mesh,
                   scratch_types=list(scratch))
    pool = pl.kernel(_sc_pool_body, out_type=(nd, nd), mesh=mesh,
                     scratch_types=list(scratch))
    return l0, l1, pool


_SC_L0, _SC_L1, _SC_POOL = _make_sc_kernels()


BR = 1000


def _tc_a_body(s_on, c_on, s_vo, c_vo, x,
               wn_on, wr_on, wn_vo, wr_vo, b, h0_o, h1_o):
    agg_on = s_on[:] / jnp.maximum(c_on[:, :1], 1.0)
    agg_vo = s_vo[:] / jnp.maximum(c_vo[:, :1], 1.0)
    h = (jnp.dot(agg_on, wn_on[:], preferred_element_type=F32)
         + jnp.dot(agg_vo, wn_vo[:], preferred_element_type=F32)
         + jnp.dot(x[:], wr_on[:] + wr_vo[:], preferred_element_type=F32)
         + b[:])
    h = jnp.maximum(h, 0.0)
    h0_o[:] = h[:, :D]
    h1_o[:] = h[:, D:]


def _tc_b_body(on0, on1, vo0, vo1, c_on, c_vo, h0, h1,
               wn_on, wr_on, wn_vo, wr_vo, b, lin_w, lin_b, h3_o):
    r_on = 1.0 / jnp.maximum(c_on[:, :1], 1.0)
    r_vo = 1.0 / jnp.maximum(c_vo[:, :1], 1.0)
    agg_on = jnp.concatenate([on0[:] * r_on, on1[:] * r_on], axis=-1)
    agg_vo = jnp.concatenate([vo0[:] * r_vo, vo1[:] * r_vo], axis=-1)
    h = jnp.concatenate([h0[:], h1[:]], axis=-1)
    z = (jnp.dot(agg_on, wn_on[:], preferred_element_type=F32)
         + jnp.dot(agg_vo, wn_vo[:], preferred_element_type=F32)
         + jnp.dot(h, wr_on[:] + wr_vo[:], preferred_element_type=F32)
         + b[:])
    z = jnp.maximum(z, 0.0)
    h3_o[:] = jnp.dot(z, lin_w[:], preferred_element_type=F32) + lin_b[:]


def _ln(x, g, b):
    m = jnp.mean(x, axis=-1, keepdims=True)
    v = jnp.mean((x - m) ** 2, axis=-1, keepdims=True)
    return (x - m) / jnp.sqrt(v + EPS) * g + b


def _tc_c_body(p0, p1, c_on, h3, norm_g, norm_b, pm_w1, pm_b1, pm_g, pm_b,
               pm_w2, pm_b2, cw1, cb1, bn_g, bn_b, bn_rm, bn_rv, cw2, cb2,
               out_o):
    pooled = (p0[:] + p1[:] + h3[:]) / jnp.maximum(c_on[:, :1], 1.0)
    h = _ln(pooled, norm_g[:], norm_b[:])
    z = jnp.maximum(jnp.dot(h, pm_w1[:], preferred_element_type=F32) + pm_b1[:], 0.0)
    z = _ln(z, pm_g[:], pm_b[:])
    z = jnp.dot(z, pm_w2[:], preferred_element_type=F32) + pm_b2[:]
    c = jnp.maximum(jnp.dot(z, cw1[:], preferred_element_type=F32) + cb1[:], 0.0)
    c = (c - bn_rm[:]) / jnp.sqrt(bn_rv[:] + EPS) * bn_g[:] + bn_b[:]
    logits = jnp.dot(c, cw2[:], preferred_element_type=F32) + cb2[:]
    m = jnp.max(logits, axis=-1, keepdims=True)
    e = jnp.exp(logits - m)
    out_o[:] = e / jnp.sum(e, axis=-1, keepdims=True)


def _row_spec(w):
    return pl.BlockSpec((BR, w), lambda i: (i, 0))


def _full_spec(shape):
    nd = len(shape)
    return pl.BlockSpec(shape, lambda i, _n=nd: (0,) * _n)


def _tc_a(s_on, c_on, s_vo, c_vo, x, wn_on, wr_on, wn_vo, wr_vo, b):
    grid = (N // BR,)
    return pl.pallas_call(
        _tc_a_body,
        grid=grid,
        in_specs=[_row_spec(D), _row_spec(D), _row_spec(D), _row_spec(D),
                  _row_spec(D), _full_spec((D, HID)), _full_spec((D, HID)),
                  _full_spec((D, HID)), _full_spec((D, HID)),
                  _full_spec((1, HID))],
        out_specs=[_row_spec(D), _row_spec(D)],
        out_shape=[jax.ShapeDtypeStruct((N, D), F32)] * 2,
    )(s_on, c_on, s_vo, c_vo, x, wn_on, wr_on, wn_vo, wr_vo, b)


def _tc_b(on0, on1, vo0, vo1, c_on, c_vo, h0, h1,
          wn_on, wr_on, wn_vo, wr_vo, b, lin_w, lin_b):
    grid = (N // BR,)
    return pl.pallas_call(
        _tc_b_body,
        grid=grid,
        in_specs=[_row_spec(D)] * 8
                 + [_full_spec((HID, HID))] * 4
                 + [_full_spec((1, HID)), _full_spec((HID, D)),
                    _full_spec((1, D))],
        out_specs=[_row_spec(D)],
        out_shape=[jax.ShapeDtypeStruct((N, D), F32)],
    )(on0, on1, vo0, vo1, c_on, c_vo, h0, h1,
      wn_on, wr_on, wn_vo, wr_vo, b, lin_w, lin_b)[0]


def _tc_c(p0, p1, c_on, h3, *w):
    grid = (N // BR,)
    wspecs = [_full_spec(a.shape) for a in w]
    return pl.pallas_call(
        _tc_c_body,
        grid=grid,
        in_specs=[_row_spec(D), _row_spec(D), _row_spec(D), _row_spec(D)]
                 + wspecs,
        out_specs=[_row_spec(OUT)],
        out_shape=[jax.ShapeDtypeStruct((N, OUT), F32)],
    )(p0, p1, c_on, h3, *w)[0]


def kernel(x_note, edge_index_onset, edge_index_voice, params):
    p = params
    src_on = edge_index_onset[0].astype(jnp.int32)
    dst_on = edge_index_onset[1].astype(jnp.int32)
    src_vo = edge_index_voice[0].astype(jnp.int32)
    dst_vo = edge_index_voice[1].astype(jnp.int32)

    epad = RT * RPC - src_on.shape[0]

    def _rs(a, fill):
        pad = jnp.full((epad,), fill, jnp.int32)
        return jnp.concatenate([a, pad]).reshape(RT, RPC)

    src_on2, dst_on2 = _rs(src_on, 0), _rs(dst_on, N)
    src_vo2, dst_vo2 = _rs(src_vo, 0), _rs(dst_vo, N)
    src_cat2 = jnp.concatenate([src_on2, src_vo2], axis=0)
    dst_cat2 = jnp.concatenate([dst_on2, dst_vo2], axis=0)

    s_on, s_vo, c_on, c_vo = _SC_L0(x_note, src_cat2, dst_cat2)

    b0 = (p['l0_on_b'] + p['l0_vo_b']).reshape(1, HID)
    h0, h1 = _tc_a(s_on, c_on, s_vo, c_vo, x_note,
                   p['l0_on_Wn'], p['l0_on_Wr'], p['l0_vo_Wn'], p['l0_vo_Wr'],
                   b0)

    on0, on1, vo0, vo1 = _SC_L1(h0, h1, src_cat2, dst_cat2)

    b1 = (p['l1_on_b'] + p['l1_vo_b']).reshape(1, HID)
    h3 = _tc_b(on0, on1, vo0, vo1, c_on, c_vo, h0, h1,
               p['l1_on_Wn'], p['l1_on_Wr'], p['l1_vo_Wn'], p['l1_vo_Wr'],
               b1, p['lin_W'], p['lin_b'].reshape(1, D))

    p0, p1 = _SC_POOL(h3, src_on2, dst_on2)

    r = lambda a: a.reshape(1, -1)
    out = _tc_c(p0, p1, c_on, h3,
                r(p['norm_g']), r(p['norm_b']),
                p['pm_W1'], r(p['pm_b1']), r(p['pm_ln_g']), r(p['pm_ln_b']),
                p['pm_W2'], r(p['pm_b2']),
                p['clf_W1'], r(p['clf_b1']),
                r(p['bn_g']), r(p['bn_b']), r(p['bn_rm']), r(p['bn_rv']),
                p['clf_W2'], r(p['clf_b2']))
    return out


# speedup vs baseline: 1.9607x; 1.9607x over previous
import jax
import jax.numpy as jnp
from jax import lax
from jax.experimental import pallas as pl
from jax.experimental.pallas import tpu as pltpu
from jax.experimental.pallas import tpu_sc as plsc

N = 10000
D = 128
HID = 256
CLF_H = 64
OUT = 3
EPS = 1e-5
F32 = jnp.float32

NS = 16
CH = 80
ROWS_PT = 624
TAIL = N - NS * ROWS_PT


def _zero_vmem(ref, nrows, width):
    z = jnp.zeros((16,), F32)

    def body(i, _):
        for k in range(width // 16):
            ref[i, pl.ds(k * 16, 16)] = z
        return 0

    lax.fori_loop(0, nrows, body, 0)


def _fill_vmem(ref, nrows, width, val):
    v = jnp.full((16,), val, F32)

    def body(i, _):
        for k in range(width // 16):
            ref[i, pl.ds(k * 16, 16)] = v
        return 0

    lax.fori_loop(0, nrows, body, 0)


def _copy_rows(src, dst, dst_base, nrows, chunk):
    full, rem = divmod(nrows, chunk)
    for t in range(full):
        pltpu.sync_copy(src, dst.at[pl.ds(dst_base + t * chunk, chunk), :])
    if rem:
        pltpu.sync_copy(src.at[pl.ds(0, rem), :],
                        dst.at[pl.ds(dst_base + full * chunk, rem), :])


def _zero_own_rows(acc, zsrc, s):
    _copy_rows(zsrc, acc, s * ROWS_PT, ROWS_PT, zsrc.shape[0])
    pl.when(s == NS - 1)(lambda: pltpu.sync_copy(
        zsrc.at[pl.ds(0, TAIL), :], acc.at[pl.ds(N - TAIL, TAIL), :]))


def _dump_own_rows(acc, out, s):
    base = s * ROWS_PT
    pltpu.sync_copy(acc.at[pl.ds(base, ROWS_PT), :],
                    out.at[pl.ds(base, ROWS_PT), :])
    pl.when(s == NS - 1)(lambda: pltpu.sync_copy(
        acc.at[pl.ds(N - TAIL, TAIL), :], out.at[pl.ds(N - TAIL, TAIL), :]))


def _stage(src, dst, sv, dv, base):
    pltpu.sync_copy(src.at[pl.ds(base, CH)], sv)
    pltpu.sync_copy(dst.at[pl.ds(base, CH)], dv)


def _seg_db(tab, src, dst, acc, sa, da, sb, db, ra, rb, sem_a, sem_b,
            edge_base, per_tile, s):
    nch = per_tile // CH
    base0 = edge_base + s * per_tile
    pairs, rem = divmod(nch, 2)

    _stage(src, dst, sa, da, base0)
    pltpu.async_copy(tab.at[sa], ra, sem_a)

    def body(i, _):
        jb = 2 * i + 1
        _stage(src, dst, sb, db, base0 + jb * CH)
        pltpu.async_copy(tab.at[sb], rb, sem_b)
        pltpu.make_async_copy(tab.at[sa], ra, sem_a).wait()
        pltpu.sync_copy(ra, acc.at[da], add=True)

        def _next():
            _stage(src, dst, sa, da, base0 + (jb + 1) * CH)
            pltpu.async_copy(tab.at[sa], ra, sem_a)

        pl.when(jb + 1 < nch)(_next)
        pltpu.make_async_copy(tab.at[sb], rb, sem_b).wait()
        pltpu.sync_copy(rb, acc.at[db], add=True)
        return 0

    lax.fori_loop(0, pairs, body, 0)
    if rem:
        pltpu.make_async_copy(tab.at[sa], ra, sem_a).wait()
        pltpu.sync_copy(ra, acc.at[da], add=True)


def _sc_l0_body(x_hbm, src_cat, dst_cat, s_on_o, s_vo_o, c_on_o, c_vo_o,
                acc, sa, da, sb, db, ra, rb, sem_a, sem_b):
    c = lax.axis_index("c")
    s = lax.axis_index("s")
    e_total = src_cat.shape[0] // 2
    per_tile = e_total // NS

    _zero_vmem(ra, CH, D)
    _zero_own_rows(acc, ra, s)
    plsc.subcore_barrier()
    _seg_db(x_hbm, src_cat, dst_cat, acc, sa, da, sb, db, ra, rb,
            sem_a, sem_b, c * e_total, per_tile, s)
    plsc.subcore_barrier()
    pl.when(c == 0)(lambda: _dump_own_rows(acc, s_on_o, s))
    pl.when(c == 1)(lambda: _dump_own_rows(acc, s_vo_o, s))

    _zero_vmem(ra, CH, D)
    _zero_own_rows(acc, ra, s)
    _fill_vmem(ra, CH, D, 1.0)
    plsc.subcore_barrier()

    def cbody(j, _):
        base = c * e_total + s * per_tile + j * CH
        pltpu.sync_copy(dst_cat.at[pl.ds(base, CH)], da)
        pltpu.sync_copy(ra, acc.at[da], add=True)
        return 0

    lax.fori_loop(0, per_tile // CH, cbody, 0)
    plsc.subcore_barrier()
    pl.when(c == 0)(lambda: _dump_own_rows(acc, c_on_o, s))
    pl.when(c == 1)(lambda: _dump_own_rows(acc, c_vo_o, s))


def _sc_l1_body(h0_hbm, h1_hbm, src_cat, dst_cat, on0_o, on1_o, vo0_o, vo1_o,
                acc, sa, da, sb, db, ra, rb, sem_a, sem_b):
    c = lax.axis_index("c")
    s = lax.axis_index("s")
    e_total = src_cat.shape[0] // 2
    per_tile = e_total // NS

    for task, (out0, out1) in enumerate(((on0_o, on1_o), (vo0_o, vo1_o))):
        _zero_vmem(ra, CH, D)
        _zero_own_rows(acc, ra, s)
        plsc.subcore_barrier()
        pl.when(c == 0)(lambda t=task: _seg_db(
            h0_hbm, src_cat, dst_cat, acc, sa, da, sb, db, ra, rb,
            sem_a, sem_b, t * e_total, per_tile, s))
        pl.when(c == 1)(lambda t=task: _seg_db(
            h1_hbm, src_cat, dst_cat, acc, sa, da, sb, db, ra, rb,
            sem_a, sem_b, t * e_total, per_tile, s))
        plsc.subcore_barrier()
        pl.when(c == 0)(lambda o=out0: _dump_own_rows(acc, o, s))
        pl.when(c == 1)(lambda o=out1: _dump_own_rows(acc, o, s))


def _sc_pool_body(h_hbm, src_on, dst_on, p0_o, p1_o,
                  acc, sa, da, sb, db, ra, rb, sem_a, sem_b):
    c = lax.axis_index("c")
    s = lax.axis_index("s")
    e_half = src_on.shape[0] // 2
    per_tile = e_half // NS

    _zero_vmem(ra, CH, D)
    _zero_own_rows(acc, ra, s)
    plsc.subcore_barrier()
    _seg_db(h_hbm, src_on, dst_on, acc, sa, da, sb, db, ra, rb,
            sem_a, sem_b, c * e_half, per_tile, s)
    plsc.subcore_barrier()
    pl.when(c == 0)(lambda: _dump_own_rows(acc, p0_o, s))
    pl.when(c == 1)(lambda: _dump_own_rows(acc, p1_o, s))


def _make_sc_kernels():
    mesh = plsc.VectorSubcoreMesh(core_axis_name="c", subcore_axis_name="s",
                                  num_cores=2, num_subcores=NS)
    f = jax.ShapeDtypeStruct
    nd = f((N, D), F32)
    i32 = jnp.int32

    scratch = [
        pltpu.VMEM_SHARED((N, D), F32),
        pltpu.VMEM((CH,), i32), pltpu.VMEM((CH,), i32),
        pltpu.VMEM((CH,), i32), pltpu.VMEM((CH,), i32),
        pltpu.VMEM((CH, D), F32), pltpu.VMEM((CH, D), F32),
        pltpu.SemaphoreType.DMA, pltpu.SemaphoreType.DMA,
    ]

    l0 = pl.kernel(_sc_l0_body, out_type=(nd, nd, nd, nd), mesh=mesh,
                   scratch_types=list(scratch))
    l1 = pl.kernel(_sc_l1_body, out_type=(nd, nd, nd, nd), mesh=mesh,
                   scratch_types=list(scratch))
    pool = pl.kernel(_sc_pool_body, out_type=(nd, nd), mesh=mesh,
                     scratch_types=list(scratch))
    return l0, l1, pool


_SC_L0, _SC_L1, _SC_POOL = _make_sc_kernels()


BR = 1000


def _tc_a_body(s_on, c_on, s_vo, c_vo, x,
               wn_on, wr_on, wn_vo, wr_vo, b, h0_o, h1_o):
    agg_on = s_on[:] / jnp.maximum(c_on[:, :1], 1.0)
    agg_vo = s_vo[:] / jnp.maximum(c_vo[:, :1], 1.0)
    h = (jnp.dot(agg_on, wn_on[:], preferred_element_type=F32)
         + jnp.dot(agg_vo, wn_vo[:], preferred_element_type=F32)
         + jnp.dot(x[:], wr_on[:] + wr_vo[:], preferred_element_type=F32)
         + b[:])
    h = jnp.maximum(h, 0.0)
    h0_o[:] = h[:, :D]
    h1_o[:] = h[:, D:]


def _tc_b_body(on0, on1, vo0, vo1, c_on, c_vo, h0, h1,
               wn_on, wr_on, wn_vo, wr_vo, b, lin_w, lin_b, h3_o):
    r_on = 1.0 / jnp.maximum(c_on[:, :1], 1.0)
    r_vo = 1.0 / jnp.maximum(c_vo[:, :1], 1.0)
    agg_on = jnp.concatenate([on0[:] * r_on, on1[:] * r_on], axis=-1)
    agg_vo = jnp.concatenate([vo0[:] * r_vo, vo1[:] * r_vo], axis=-1)
    h = jnp.concatenate([h0[:], h1[:]], axis=-1)
    z = (jnp.dot(agg_on, wn_on[:], preferred_element_type=F32)
         + jnp.dot(agg_vo, wn_vo[:], preferred_element_type=F32)
         + jnp.dot(h, wr_on[:] + wr_vo[:], preferred_element_type=F32)
         + b[:])
    z = jnp.maximum(z, 0.0)
    h3_o[:] = jnp.dot(z, lin_w[:], preferred_element_type=F32) + lin_b[:]


def _ln(x, g, b):
    m = jnp.mean(x, axis=-1, keepdims=True)
    v = jnp.mean((x - m) ** 2, axis=-1, keepdims=True)
    return (x - m) / jnp.sqrt(v + EPS) * g + b


def _tc_c_body(p0, p1, c_on, h3, norm_g, norm_b, pm_w1, pm_b1, pm_g, pm_b,
               pm_w2, pm_b2, cw1, cb1, bn_g, bn_b, bn_rm, bn_rv, cw2, cb2,
               out_o):
    pooled = (p0[:] + p1[:] + h3[:]) / jnp.maximum(c_on[:, :1], 1.0)
    h = _ln(pooled, norm_g[:], norm_b[:])
    z = jnp.maximum(jnp.dot(h, pm_w1[:], preferred_element_type=F32) + pm_b1[:], 0.0)
    z = _ln(z, pm_g[:], pm_b[:])
    z = jnp.dot(z, pm_w2[:], preferred_element_type=F32) + pm_b2[:]
    c = jnp.maximum(jnp.dot(z, cw1[:], preferred_element_type=F32) + cb1[:], 0.0)
    c = (c - bn_rm[:]) / jnp.sqrt(bn_rv[:] + EPS) * bn_g[:] + bn_b[:]
    logits = jnp.dot(c, cw2[:], preferred_element_type=F32) + cb2[:]
    m = jnp.max(logits, axis=-1, keepdims=True)
    e = jnp.exp(logits - m)
    out_o[:] = e / jnp.sum(e, axis=-1, keepdims=True)


def _row_spec(w):
    return pl.BlockSpec((BR, w), lambda i: (i, 0))


def _full_spec(shape):
    nd = len(shape)
    return pl.BlockSpec(shape, lambda i, _n=nd: (0,) * _n)


def _tc_a(s_on, c_on, s_vo, c_vo, x, wn_on, wr_on, wn_vo, wr_vo, b):
    grid = (N // BR,)
    return pl.pallas_call(
        _tc_a_body,
        grid=grid,
        in_specs=[_row_spec(D), _row_spec(D), _row_spec(D), _row_spec(D),
                  _row_spec(D), _full_spec((D, HID)), _full_spec((D, HID)),
                  _full_spec((D, HID)), _full_spec((D, HID)),
                  _full_spec((1, HID))],
        out_specs=[_row_spec(D), _row_spec(D)],
        out_shape=[jax.ShapeDtypeStruct((N, D), F32)] * 2,
    )(s_on, c_on, s_vo, c_vo, x, wn_on, wr_on, wn_vo, wr_vo, b)


def _tc_b(on0, on1, vo0, vo1, c_on, c_vo, h0, h1,
          wn_on, wr_on, wn_vo, wr_vo, b, lin_w, lin_b):
    grid = (N // BR,)
    return pl.pallas_call(
        _tc_b_body,
        grid=grid,
        in_specs=[_row_spec(D)] * 8
                 + [_full_spec((HID, HID))] * 4
                 + [_full_spec((1, HID)), _full_spec((HID, D)),
                    _full_spec((1, D))],
        out_specs=[_row_spec(D)],
        out_shape=[jax.ShapeDtypeStruct((N, D), F32)],
    )(on0, on1, vo0, vo1, c_on, c_vo, h0, h1,
      wn_on, wr_on, wn_vo, wr_vo, b, lin_w, lin_b)[0]


def _tc_c(p0, p1, c_on, h3, *w):
    grid = (N // BR,)
    wspecs = [_full_spec(a.shape) for a in w]
    return pl.pallas_call(
        _tc_c_body,
        grid=grid,
        in_specs=[_row_spec(D), _row_spec(D), _row_spec(D), _row_spec(D)]
                 + wspecs,
        out_specs=[_row_spec(OUT)],
        out_shape=[jax.ShapeDtypeStruct((N, OUT), F32)],
    )(p0, p1, c_on, h3, *w)[0]


def kernel(x_note, edge_index_onset, edge_index_voice, params):
    p = params
    src_on = edge_index_onset[0].astype(jnp.int32)
    dst_on = edge_index_onset[1].astype(jnp.int32)
    src_vo = edge_index_voice[0].astype(jnp.int32)
    dst_vo = edge_index_voice[1].astype(jnp.int32)

    src_cat = jnp.concatenate([src_on, src_vo])
    dst_cat = jnp.concatenate([dst_on, dst_vo])
    s_on, s_vo, c_on, c_vo = _SC_L0(x_note, src_cat, dst_cat)

    b0 = (p['l0_on_b'] + p['l0_vo_b']).reshape(1, HID)
    h0, h1 = _tc_a(s_on, c_on, s_vo, c_vo, x_note,
                   p['l0_on_Wn'], p['l0_on_Wr'], p['l0_vo_Wn'], p['l0_vo_Wr'],
                   b0)

    on0, on1, vo0, vo1 = _SC_L1(h0, h1, src_cat, dst_cat)

    b1 = (p['l1_on_b'] + p['l1_vo_b']).reshape(1, HID)
    h3 = _tc_b(on0, on1, vo0, vo1, c_on, c_vo, h0, h1,
               p['l1_on_Wn'], p['l1_on_Wr'], p['l1_vo_Wn'], p['l1_vo_Wr'],
               b1, p['lin_W'], p['lin_b'].reshape(1, D))

    p0, p1 = _SC_POOL(h3, src_on, dst_on)

    r = lambda a: a.reshape(1, -1)
    out = _tc_c(p0, p1, c_on, h3,
                r(p['norm_g']), r(p['norm_b']),
                p['pm_W1'], r(p['pm_b1']), r(p['pm_ln_g']), r(p['pm_ln_b']),
                p['pm_W2'], r(p['pm_b2']),
                p['clf_W1'], r(p['clf_b1']),
                r(p['bn_g']), r(p['bn_b']), r(p['bn_rm']), r(p['bn_rv']),
                p['clf_W2'], r(p['clf_b2']))
    return out


# speedup vs baseline: 2.0983x; 1.0702x over previous
import jax
import jax.numpy as jnp
from jax import lax
from jax.experimental import pallas as pl
from jax.experimental.pallas import tpu as pltpu
from jax.experimental.pallas import tpu_sc as plsc

N = 10000
D = 128
HID = 256
CLF_H = 64
OUT = 3
EPS = 1e-5
F32 = jnp.float32

NS = 16
CH = 80
ROWS_PT = 624
TAIL = N - NS * ROWS_PT


def _zero_vmem(ref, nrows, width):
    z = jnp.zeros((16,), F32)

    def body(i, _):
        for k in range(width // 16):
            ref[i, pl.ds(k * 16, 16)] = z
        return 0

    lax.fori_loop(0, nrows, body, 0)


def _fill_vmem(ref, nrows, width, val):
    v = jnp.full((16,), val, F32)

    def body(i, _):
        for k in range(width // 16):
            ref[i, pl.ds(k * 16, 16)] = v
        return 0

    lax.fori_loop(0, nrows, body, 0)


def _copy_rows(src, dst, dst_base, nrows, chunk):
    full, rem = divmod(nrows, chunk)
    for t in range(full):
        pltpu.sync_copy(src, dst.at[pl.ds(dst_base + t * chunk, chunk), :])
    if rem:
        pltpu.sync_copy(src.at[pl.ds(0, rem), :],
                        dst.at[pl.ds(dst_base + full * chunk, rem), :])


def _zero_own_rows(acc, zsrc, s):
    _copy_rows(zsrc, acc, s * ROWS_PT, ROWS_PT, zsrc.shape[0])
    pl.when(s == NS - 1)(lambda: pltpu.sync_copy(
        zsrc.at[pl.ds(0, TAIL), :], acc.at[pl.ds(N - TAIL, TAIL), :]))


def _dump_own_rows(acc, out, s):
    base = s * ROWS_PT
    pltpu.sync_copy(acc.at[pl.ds(base, ROWS_PT), :],
                    out.at[pl.ds(base, ROWS_PT), :])
    pl.when(s == NS - 1)(lambda: pltpu.sync_copy(
        acc.at[pl.ds(N - TAIL, TAIL), :], out.at[pl.ds(N - TAIL, TAIL), :]))


def _stage(src, dst, sv, dv, base):
    pltpu.sync_copy(src.at[pl.ds(base, CH)], sv)
    pltpu.sync_copy(dst.at[pl.ds(base, CH)], dv)


def _seg_db(tab, src, dst, acc, sa, da, sb, db, ra, rb, sem_a, sem_b,
            edge_base, per_tile, s):
    nch = per_tile // CH
    base0 = edge_base + s * per_tile
    pairs, rem = divmod(nch, 2)

    _stage(src, dst, sa, da, base0)
    pltpu.async_copy(tab.at[sa], ra, sem_a)

    def body(i, _):
        jb = 2 * i + 1
        _stage(src, dst, sb, db, base0 + jb * CH)
        pltpu.async_copy(tab.at[sb], rb, sem_b)
        pltpu.make_async_copy(tab.at[sa], ra, sem_a).wait()
        pltpu.sync_copy(ra, acc.at[da], add=True)

        def _next():
            _stage(src, dst, sa, da, base0 + (jb + 1) * CH)
            pltpu.async_copy(tab.at[sa], ra, sem_a)

        pl.when(jb + 1 < nch)(_next)
        pltpu.make_async_copy(tab.at[sb], rb, sem_b).wait()
        pltpu.sync_copy(rb, acc.at[db], add=True)
        return 0

    lax.fori_loop(0, pairs, body, 0)
    if rem:
        pltpu.make_async_copy(tab.at[sa], ra, sem_a).wait()
        pltpu.sync_copy(ra, acc.at[da], add=True)


def _sc_l0_body(x_hbm, src_cat, dst_cat, s_on_o, s_vo_o, c_on_o, c_vo_o,
                acc, sa, da, sb, db, ra, rb, sem_a, sem_b):
    c = lax.axis_index("c")
    s = lax.axis_index("s")
    e_total = src_cat.shape[0] // 2
    per_tile = e_total // NS

    _zero_vmem(ra, CH, D)
    _zero_own_rows(acc, ra, s)
    plsc.subcore_barrier()
    _seg_db(x_hbm, src_cat, dst_cat, acc, sa, da, sb, db, ra, rb,
            sem_a, sem_b, c * e_total, per_tile, s)
    plsc.subcore_barrier()
    pl.when(c == 0)(lambda: _dump_own_rows(acc, s_on_o, s))
    pl.when(c == 1)(lambda: _dump_own_rows(acc, s_vo_o, s))

    _zero_vmem(ra, CH, D)
    _zero_own_rows(acc, ra, s)
    _fill_vmem(ra, CH, D, 1.0)
    plsc.subcore_barrier()

    nch = per_tile // CH
    base0 = c * e_total + s * per_tile
    pltpu.async_copy(dst_cat.at[pl.ds(base0, CH)], da, sem_a)

    def cbody(i, _):
        jb = 2 * i + 1
        pltpu.async_copy(dst_cat.at[pl.ds(base0 + jb * CH, CH)], db, sem_b)
        pltpu.make_async_copy(dst_cat.at[pl.ds(base0, CH)], da, sem_a).wait()
        pltpu.sync_copy(ra, acc.at[da], add=True)

        def _next():
            pltpu.async_copy(
                dst_cat.at[pl.ds(base0 + (jb + 1) * CH, CH)], da, sem_a)

        pl.when(jb + 1 < nch)(_next)
        pltpu.make_async_copy(dst_cat.at[pl.ds(base0, CH)], db, sem_b).wait()
        pltpu.sync_copy(ra, acc.at[db], add=True)
        return 0

    lax.fori_loop(0, nch // 2, cbody, 0)
    plsc.subcore_barrier()
    pl.when(c == 0)(lambda: _dump_own_rows(acc, c_on_o, s))
    pl.when(c == 1)(lambda: _dump_own_rows(acc, c_vo_o, s))


def _sc_l1_body(h0_hbm, h1_hbm, src_cat, dst_cat, on0_o, on1_o, vo0_o, vo1_o,
                acc, sa, da, sb, db, ra, rb, sem_a, sem_b):
    c = lax.axis_index("c")
    s = lax.axis_index("s")
    e_total = src_cat.shape[0] // 2
    per_tile = e_total // NS

    for task, (out0, out1) in enumerate(((on0_o, on1_o), (vo0_o, vo1_o))):
        _zero_vmem(ra, CH, D)
        _zero_own_rows(acc, ra, s)
        plsc.subcore_barrier()
        pl.when(c == 0)(lambda t=task: _seg_db(
            h0_hbm, src_cat, dst_cat, acc, sa, da, sb, db, ra, rb,
            sem_a, sem_b, t * e_total, per_tile, s))
        pl.when(c == 1)(lambda t=task: _seg_db(
            h1_hbm, src_cat, dst_cat, acc, sa, da, sb, db, ra, rb,
            sem_a, sem_b, t * e_total, per_tile, s))
        plsc.subcore_barrier()
        pl.when(c == 0)(lambda o=out0: _dump_own_rows(acc, o, s))
        pl.when(c == 1)(lambda o=out1: _dump_own_rows(acc, o, s))


def _sc_pool_body(h_hbm, src_on, dst_on, p0_o, p1_o,
                  acc, sa, da, sb, db, ra, rb, sem_a, sem_b):
    c = lax.axis_index("c")
    s = lax.axis_index("s")
    e_half = src_on.shape[0] // 2
    per_tile = e_half // NS

    _zero_vmem(ra, CH, D)
    _zero_own_rows(acc, ra, s)
    plsc.subcore_barrier()
    _seg_db(h_hbm, src_on, dst_on, acc, sa, da, sb, db, ra, rb,
            sem_a, sem_b, c * e_half, per_tile, s)
    plsc.subcore_barrier()
    pl.when(c == 0)(lambda: _dump_own_rows(acc, p0_o, s))
    pl.when(c == 1)(lambda: _dump_own_rows(acc, p1_o, s))


def _make_sc_kernels():
    mesh = plsc.VectorSubcoreMesh(core_axis_name="c", subcore_axis_name="s",
                                  num_cores=2, num_subcores=NS)
    f = jax.ShapeDtypeStruct
    nd = f((N, D), F32)
    i32 = jnp.int32

    scratch = [
        pltpu.VMEM_SHARED((N, D), F32),
        pltpu.VMEM((CH,), i32), pltpu.VMEM((CH,), i32),
        pltpu.VMEM((CH,), i32), pltpu.VMEM((CH,), i32),
        pltpu.VMEM((CH, D), F32), pltpu.VMEM((CH, D), F32),
        pltpu.SemaphoreType.DMA, pltpu.SemaphoreType.DMA,
    ]

    l0 = pl.kernel(_sc_l0_body, out_type=(nd, nd, nd, nd), mesh=mesh,
                   scratch_types=list(scratch))
    l1 = pl.kernel(_sc_l1_body, out_type=(nd, nd, nd, nd), mesh=mesh,
                   scratch_types=list(scratch))
    pool = pl.kernel(_sc_pool_body, out_type=(nd, nd), mesh=mesh,
                     scratch_types=list(scratch))
    return l0, l1, pool


_SC_L0, _SC_L1, _SC_POOL = _make_sc_kernels()


BR = 1000


def _tc_a_body(s_on, c_on, s_vo, c_vo, x,
               wn_on, wr_on, wn_vo, wr_vo, b, h0_o, h1_o):
    agg_on = s_on[:] / jnp.maximum(c_on[:, :1], 1.0)
    agg_vo = s_vo[:] / jnp.maximum(c_vo[:, :1], 1.0)
    h = (jnp.dot(agg_on, wn_on[:], preferred_element_type=F32)
         + jnp.dot(agg_vo, wn_vo[:], preferred_element_type=F32)
         + jnp.dot(x[:], wr_on[:] + wr_vo[:], preferred_element_type=F32)
         + b[:])
    h = jnp.maximum(h, 0.0)
    h0_o[:] = h[:, :D]
    h1_o[:] = h[:, D:]


def _tc_b_body(on0, on1, vo0, vo1, c_on, c_vo, h0, h1,
               wn_on, wr_on, wn_vo, wr_vo, b, lin_w, lin_b, h3_o):
    r_on = 1.0 / jnp.maximum(c_on[:, :1], 1.0)
    r_vo = 1.0 / jnp.maximum(c_vo[:, :1], 1.0)
    agg_on = jnp.concatenate([on0[:] * r_on, on1[:] * r_on], axis=-1)
    agg_vo = jnp.concatenate([vo0[:] * r_vo, vo1[:] * r_vo], axis=-1)
    h = jnp.concatenate([h0[:], h1[:]], axis=-1)
    z = (jnp.dot(agg_on, wn_on[:], preferred_element_type=F32)
         + jnp.dot(agg_vo, wn_vo[:], preferred_element_type=F32)
         + jnp.dot(h, wr_on[:] + wr_vo[:], preferred_element_type=F32)
         + b[:])
    z = jnp.maximum(z, 0.0)
    h3_o[:] = jnp.dot(z, lin_w[:], preferred_element_type=F32) + lin_b[:]


def _ln(x, g, b):
    m = jnp.mean(x, axis=-1, keepdims=True)
    v = jnp.mean((x - m) ** 2, axis=-1, keepdims=True)
    return (x - m) / jnp.sqrt(v + EPS) * g + b


def _tc_c_body(p0, p1, c_on, h3, norm_g, norm_b, pm_w1, pm_b1, pm_g, pm_b,
               pm_w2, pm_b2, cw1, cb1, bn_g, bn_b, bn_rm, bn_rv, cw2, cb2,
               out_o):
    pooled = (p0[:] + p1[:] + h3[:]) / jnp.maximum(c_on[:, :1], 1.0)
    h = _ln(pooled, norm_g[:], norm_b[:])
    z = jnp.maximum(jnp.dot(h, pm_w1[:], preferred_element_type=F32) + pm_b1[:], 0.0)
    z = _ln(z, pm_g[:], pm_b[:])
    z = jnp.dot(z, pm_w2[:], preferred_element_type=F32) + pm_b2[:]
    c = jnp.maximum(jnp.dot(z, cw1[:], preferred_element_type=F32) + cb1[:], 0.0)
    c = (c - bn_rm[:]) / jnp.sqrt(bn_rv[:] + EPS) * bn_g[:] + bn_b[:]
    logits = jnp.dot(c, cw2[:], preferred_element_type=F32) + cb2[:]
    m = jnp.max(logits, axis=-1, keepdims=True)
    e = jnp.exp(logits - m)
    out_o[:] = e / jnp.sum(e, axis=-1, keepdims=True)


def _row_spec(w):
    return pl.BlockSpec((BR, w), lambda i: (i, 0))


def _full_spec(shape):
    nd = len(shape)
    return pl.BlockSpec(shape, lambda i, _n=nd: (0,) * _n)


def _tc_a(s_on, c_on, s_vo, c_vo, x, wn_on, wr_on, wn_vo, wr_vo, b):
    grid = (N // BR,)
    return pl.pallas_call(
        _tc_a_body,
        grid=grid,
        in_specs=[_row_spec(D), _row_spec(D), _row_spec(D), _row_spec(D),
                  _row_spec(D), _full_spec((D, HID)), _full_spec((D, HID)),
                  _full_spec((D, HID)), _full_spec((D, HID)),
                  _full_spec((1, HID))],
        out_specs=[_row_spec(D), _row_spec(D)],
        out_shape=[jax.ShapeDtypeStruct((N, D), F32)] * 2,
    )(s_on, c_on, s_vo, c_vo, x, wn_on, wr_on, wn_vo, wr_vo, b)


def _tc_b(on0, on1, vo0, vo1, c_on, c_vo, h0, h1,
          wn_on, wr_on, wn_vo, wr_vo, b, lin_w, lin_b):
    grid = (N // BR,)
    return pl.pallas_call(
        _tc_b_body,
        grid=grid,
        in_specs=[_row_spec(D)] * 8
                 + [_full_spec((HID, HID))] * 4
                 + [_full_spec((1, HID)), _full_spec((HID, D)),
                    _full_spec((1, D))],
        out_specs=[_row_spec(D)],
        out_shape=[jax.ShapeDtypeStruct((N, D), F32)],
    )(on0, on1, vo0, vo1, c_on, c_vo, h0, h1,
      wn_on, wr_on, wn_vo, wr_vo, b, lin_w, lin_b)[0]


def _tc_c(p0, p1, c_on, h3, *w):
    grid = (N // BR,)
    wspecs = [_full_spec(a.shape) for a in w]
    return pl.pallas_call(
        _tc_c_body,
        grid=grid,
        in_specs=[_row_spec(D), _row_spec(D), _row_spec(D), _row_spec(D)]
                 + wspecs,
        out_specs=[_row_spec(OUT)],
        out_shape=[jax.ShapeDtypeStruct((N, OUT), F32)],
    )(p0, p1, c_on, h3, *w)[0]


def kernel(x_note, edge_index_onset, edge_index_voice, params):
    p = params
    src_on = edge_index_onset[0].astype(jnp.int32)
    dst_on = edge_index_onset[1].astype(jnp.int32)
    src_vo = edge_index_voice[0].astype(jnp.int32)
    dst_vo = edge_index_voice[1].astype(jnp.int32)

    src_cat = jnp.concatenate([src_on, src_vo])
    dst_cat = jnp.concatenate([dst_on, dst_vo])
    s_on, s_vo, c_on, c_vo = _SC_L0(x_note, src_cat, dst_cat)

    b0 = (p['l0_on_b'] + p['l0_vo_b']).reshape(1, HID)
    h0, h1 = _tc_a(s_on, c_on, s_vo, c_vo, x_note,
                   p['l0_on_Wn'], p['l0_on_Wr'], p['l0_vo_Wn'], p['l0_vo_Wr'],
                   b0)

    on0, on1, vo0, vo1 = _SC_L1(h0, h1, src_cat, dst_cat)

    b1 = (p['l1_on_b'] + p['l1_vo_b']).reshape(1, HID)
    h3 = _tc_b(on0, on1, vo0, vo1, c_on, c_vo, h0, h1,
               p['l1_on_Wn'], p['l1_on_Wr'], p['l1_vo_Wn'], p['l1_vo_Wr'],
               b1, p['lin_W'], p['lin_b'].reshape(1, D))

    p0, p1 = _SC_POOL(h3, src_on, dst_on)

    r = lambda a: a.reshape(1, -1)
    out = _tc_c(p0, p1, c_on, h3,
                r(p['norm_g']), r(p['norm_b']),
                p['pm_W1'], r(p['pm_b1']), r(p['pm_ln_g']), r(p['pm_ln_b']),
                p['pm_W2'], r(p['pm_b2']),
                p['clf_W1'], r(p['clf_b1']),
                r(p['bn_g']), r(p['bn_b']), r(p['bn_rm']), r(p['bn_rv']),
                p['clf_W2'], r(p['clf_b2']))
    return out
